# baseline (speedup 1.0000x reference)
import functools

import jax
import jax.numpy as jnp
from jax import lax
from jax.experimental import pallas as pl
from jax.experimental.pallas import tpu as pltpu
from jax.experimental.pallas import tpu_sc as plsc


@functools.lru_cache(maxsize=None)
def _make_sc_lookup(B, D, dtype):
    mesh = plsc.VectorSubcoreMesh(core_axis_name="c", subcore_axis_name="s")

    @functools.partial(
        pl.kernel,
        mesh=mesh,
        out_type=jax.ShapeDtypeStruct((B, D), dtype),
    )
    def k(w_hbm, out_hbm):
        cid = lax.axis_index("c")
        sid = lax.axis_index("s")

        @pl.when((cid == 0) & (sid == 0))
        def _():
            pltpu.sync_copy(w_hbm.at[pl.ds(0, B)], out_hbm)

    return k


def kernel(tokens, W_pos):
    B = tokens.shape[0]
    D = W_pos.shape[1]
    return _make_sc_lookup(B, D, W_pos.dtype)(W_pos)


# speedup vs baseline: 1.1747x; 1.1747x over previous
import functools

import jax
import jax.numpy as jnp
from jax import lax
from jax.experimental import pallas as pl
from jax.experimental.pallas import tpu as pltpu
from jax.experimental.pallas import tpu_sc as plsc


@functools.lru_cache(maxsize=None)
def _make_sc_lookup(B, D, dtype):
    mesh = plsc.ScalarSubcoreMesh(axis_name="c", num_cores=1)

    @functools.partial(
        pl.kernel,
        mesh=mesh,
        out_type=jax.ShapeDtypeStruct((B, D), dtype),
    )
    def k(w_hbm, out_hbm):
        pltpu.sync_copy(w_hbm.at[pl.ds(0, B)], out_hbm)

    return k


def kernel(tokens, W_pos):
    B = tokens.shape[0]
    D = W_pos.shape[1]
    return _make_sc_lookup(B, D, W_pos.dtype)(W_pos)


# speedup vs baseline: 1.1752x; 1.0004x over previous
import functools

import jax
import jax.numpy as jnp
from jax import lax
from jax.experimental import pallas as pl
from jax.experimental.pallas import tpu as pltpu
from jax.experimental.pallas import tpu_sc as plsc


@functools.lru_cache(maxsize=None)
def _make_sc_lookup(B, D, dtype):
    mesh = plsc.ScalarSubcoreMesh(axis_name="c", num_cores=1)

    @functools.partial(
        pl.kernel,
        mesh=mesh,
        out_type=jax.ShapeDtypeStruct((B, D), dtype),
        compiler_params=pltpu.CompilerParams(
            skip_device_barrier=True,
            disable_bounds_checks=True,
            disable_semaphore_checks=True,
        ),
    )
    def k(w_hbm, out_hbm):
        pltpu.sync_copy(w_hbm.at[pl.ds(0, B)], out_hbm)

    return k


def kernel(tokens, W_pos):
    B = tokens.shape[0]
    D = W_pos.shape[1]
    return _make_sc_lookup(B, D, W_pos.dtype)(W_pos)


# speedup vs baseline: 1.1757x; 1.0004x over previous
import functools

import jax
import jax.numpy as jnp
from jax import lax
from jax.experimental import pallas as pl
from jax.experimental.pallas import tpu as pltpu
from jax.experimental.pallas import tpu_sc as plsc


@functools.lru_cache(maxsize=None)
def _make_sc_lookup(B, D, dtype):
    mesh = plsc.ScalarSubcoreMesh(axis_name="c", num_cores=1)

    @functools.partial(
        pl.kernel,
        mesh=mesh,
        out_type=jax.ShapeDtypeStruct((B, D), dtype),
    )
    def k(w_hbm, out_hbm):
        pltpu.sync_copy(w_hbm.at[pl.ds(0, B)], out_hbm)

    return k


def kernel(tokens, W_pos):
    B = tokens.shape[0]
    D = W_pos.shape[1]
    return _make_sc_lookup(B, D, W_pos.dtype)(W_pos)
